# baseline (speedup 1.0000x reference)
import functools

import jax
import jax.numpy as jnp
from jax import lax
from jax.experimental import pallas as pl
from jax.experimental.pallas import tpu as pltpu
from jax.experimental.pallas import tpu_sc as plsc

B = 8
N = 2048
C = 64
S = 512
NS = 32
RADIUS2 = 0.2 ** 2
H0 = 64

SC_CORES = 2
SC_SUBCORES = 16
SC_WORKERS = SC_CORES * SC_SUBCORES
ROWS_TOTAL = B * S * NS
ROWS_PER_W = ROWS_TOTAL // SC_WORKERS
CHUNK = 128
CHUNKS_PER_W = ROWS_PER_W // CHUNK


def _row_sum(a):
    w = a.shape[1]
    while w > 128:
        h = w // 2
        a = a[:, :h] + a[:, h:]
        w = h
    return jnp.sum(a, axis=1, keepdims=True)


def _row_max(a):
    w = a.shape[1]
    while w > 128:
        h = w // 2
        a = jnp.maximum(a[:, :h], a[:, h:])
        w = h
    return jnp.max(a, axis=1, keepdims=True)


def _row_min(a):
    w = a.shape[1]
    while w > 128:
        h = w // 2
        a = jnp.minimum(a[:, :h], a[:, h:])
        w = h
    return jnp.min(a, axis=1, keepdims=True)


def _fps_body(xt_ref, nx_ref):
    x = xt_ref[0]
    y = xt_ref[1]
    z = xt_ref[2]
    col = lax.broadcasted_iota(jnp.int32, (B, N), 1)
    col_s = lax.broadcasted_iota(jnp.int32, (B, S), 1)

    def step(i, carry):
        distance, far, ax, ay, az = carry
        onehot = col == far
        cx = _row_sum(jnp.where(onehot, x, 0.0))
        cy = _row_sum(jnp.where(onehot, y, 0.0))
        cz = _row_sum(jnp.where(onehot, z, 0.0))
        sel = col_s == i
        ax = jnp.where(sel, cx, ax)
        ay = jnp.where(sel, cy, ay)
        az = jnp.where(sel, cz, az)
        dx = x - cx
        dy = y - cy
        dz = z - cz
        dist = dx * dx + dy * dy + dz * dz
        distance = jnp.minimum(distance, dist)
        rowmax = _row_max(distance)
        far = _row_min(jnp.where(distance == rowmax, col, N)).astype(jnp.int32)
        return distance, far, ax, ay, az

    init = (jnp.full((B, N), 1e10, dtype=jnp.float32),
            jnp.zeros((B, 1), dtype=jnp.int32),
            jnp.zeros((B, S), dtype=jnp.float32),
            jnp.zeros((B, S), dtype=jnp.float32),
            jnp.zeros((B, S), dtype=jnp.float32))
    _, _, ax, ay, az = lax.fori_loop(0, S, step, init)
    nx_ref[0] = ax
    nx_ref[1] = ay
    nx_ref[2] = az


def _fps_call(xt):
    return pl.pallas_call(
        _fps_body,
        grid=(1,),
        in_specs=[pl.BlockSpec((3, B, N), lambda i: (0, 0, 0))],
        out_specs=pl.BlockSpec((3, B, S), lambda i: (0, 0, 0)),
        out_shape=jax.ShapeDtypeStruct((3, B, S), jnp.float32),
    )(xt)


S_BLK = 512


def _bq_body(b0, xt_ref, nxz_ref, out_ref):
    b = pl.program_id(0) + b0
    px = xt_ref[0, pl.dslice(b, 1), :]
    py = xt_ref[1, pl.dslice(b, 1), :]
    pz = xt_ref[2, pl.dslice(b, 1), :]
    nblk = nxz_ref[0]
    cx = nblk[:, 0:1]
    cy = nblk[:, 1:2]
    cz = nblk[:, 2:3]
    dx = px - cx
    dy = py - cy
    dz = pz - cz
    d2 = dx * dx + dy * dy + dz * dz
    col = lax.broadcasted_iota(jnp.int32, (S_BLK, N), 1)
    cand_base = jnp.where(d2 <= RADIUS2, col, N)
    base = b * N

    def row_min_native(a):
        w = a.shape[1]
        while w > 128:
            h = w // 2
            a = jnp.minimum(a[:, :h], a[:, h:])
            w = h
        return jnp.min(a, axis=1, keepdims=True)

    first = row_min_native(cand_base)
    out_ref[0, :, 0:1] = first + base
    prev = first
    for k in range(1, NS):
        cand = jnp.where(col > prev, cand_base, N)
        m = row_min_native(cand)
        out_ref[0, :, k:k + 1] = jnp.where(m == N, first, m) + base
        prev = m


def _bq_call(xt, new_xyz, b0, nb):
    return pl.pallas_call(
        functools.partial(_bq_body, b0),
        grid=(nb, S // S_BLK),
        in_specs=[
            pl.BlockSpec((3, B, N), lambda b, sb: (0, 0, 0)),
            pl.BlockSpec((1, S_BLK, 3), lambda b, sb: (b + b0, sb, 0)),
        ],
        out_specs=pl.BlockSpec((1, S_BLK, NS), lambda b, sb: (b, sb, 0)),
        out_shape=jax.ShapeDtypeStruct((nb, S, NS), jnp.int32),
    )(xt, new_xyz)


def _t0_body(p_ref, x_ref, wp_ref, wx_ref, out_ref):
    out_ref[...] = (
        jnp.dot(p_ref[0], wp_ref[...], preferred_element_type=jnp.float32)
        + jnp.dot(x_ref[0], wx_ref[...], preferred_element_type=jnp.float32))


def _t0_call(points, xyz, w_pts, w_xyz):
    return pl.pallas_call(
        _t0_body,
        grid=(B,),
        in_specs=[
            pl.BlockSpec((1, N, C), lambda b: (b, 0, 0)),
            pl.BlockSpec((1, N, 3), lambda b: (b, 0, 0)),
            pl.BlockSpec((C, H0), lambda b: (0, 0)),
            pl.BlockSpec((3, H0), lambda b: (0, 0)),
        ],
        out_specs=pl.BlockSpec((N, H0), lambda b: (b, 0)),
        out_shape=jax.ShapeDtypeStruct((B * N, H0), jnp.float32),
    )(points, xyz, w_pts, w_xyz)


def _sc_gather_call(idx3, table):
    nchunks = idx3.shape[1]
    nrows = SC_WORKERS * nchunks * CHUNK

    @functools.partial(
        pl.kernel,
        out_type=jax.ShapeDtypeStruct((nrows, H0), jnp.float32),
        mesh=plsc.VectorSubcoreMesh(core_axis_name="c", subcore_axis_name="s"),
        compiler_params=pltpu.CompilerParams(use_tc_tiling_on_sc=False),
        scratch_types=[
            pltpu.VMEM((nchunks, CHUNK), jnp.int32),
            pltpu.VMEM((CHUNK, H0), jnp.float32),
            pltpu.VMEM((CHUNK, H0), jnp.float32),
            pltpu.SemaphoreType.DMA,
            pltpu.SemaphoreType.DMA,
        ],
    )
    def k(idx_hbm, table_hbm, out_hbm, idx_v, buf0, buf1, sem0, sem1):
        wid = lax.axis_index("s") * SC_CORES + lax.axis_index("c")
        pltpu.sync_copy(idx_hbm.at[wid], idx_v)
        row0 = wid * nchunks * CHUNK

        pltpu.async_copy(table_hbm.at[idx_v.at[0]], buf0, sem0)

        def pair_body(jh, _):
            j0 = 2 * jh
            pltpu.async_copy(table_hbm.at[idx_v.at[j0 + 1]], buf1, sem1)
            pltpu.make_async_copy(table_hbm.at[pl.dslice(0, CHUNK)], buf0,
                                  sem0).wait()
            pltpu.sync_copy(buf0, out_hbm.at[pl.dslice(row0 + j0 * CHUNK,
                                                       CHUNK)])

            @pl.when(jh < nchunks // 2 - 1)
            def _():
                pltpu.async_copy(table_hbm.at[idx_v.at[j0 + 2]], buf0, sem0)

            pltpu.make_async_copy(table_hbm.at[pl.dslice(0, CHUNK)], buf1,
                                  sem1).wait()
            pltpu.sync_copy(buf1, out_hbm.at[pl.dslice(row0 + (j0 + 1) * CHUNK,
                                                       CHUNK)])
            return 0

        lax.fori_loop(0, nchunks // 2, pair_body, 0)

    return k(idx3, table)


M_BLK = 128
MR = M_BLK * NS


def _mlp_body(g_ref, nxz_ref, wx_ref, b0_ref, w1_ref, b1_ref, w2_ref, b2_ref,
              out_ref):
    xr = g_ref[...]
    nxz = nxz_ref[...]
    corr = jnp.dot(nxz, wx_ref[...],
                   preferred_element_type=jnp.float32) - b0_ref[...]
    h = (xr.reshape(M_BLK, NS, H0) - corr[:, None, :]).reshape(MR, H0)
    h = jnp.maximum(h, 0.0)
    h = jnp.dot(h, w1_ref[...], preferred_element_type=jnp.float32)
    h = jnp.maximum(h + b1_ref[...], 0.0)
    h = jnp.dot(h, w2_ref[...], preferred_element_type=jnp.float32)
    h = jnp.maximum(h + b2_ref[...], 0.0)
    out_ref[...] = jnp.max(h.reshape(M_BLK, NS, 128), axis=1)


def _mlp_call(grouped, nxf, w_xyz, b0, w1, b1, w2, b2):
    nblk = grouped.shape[0] // MR
    return pl.pallas_call(
        _mlp_body,
        grid=(nblk,),
        in_specs=[
            pl.BlockSpec((MR, H0), lambda g: (g, 0)),
            pl.BlockSpec((M_BLK, 3), lambda g: (g, 0)),
            pl.BlockSpec((3, H0), lambda g: (0, 0)),
            pl.BlockSpec((1, 64), lambda g: (0, 0)),
            pl.BlockSpec((64, 64), lambda g: (0, 0)),
            pl.BlockSpec((1, 64), lambda g: (0, 0)),
            pl.BlockSpec((64, 128), lambda g: (0, 0)),
            pl.BlockSpec((1, 128), lambda g: (0, 0)),
        ],
        out_specs=pl.BlockSpec((M_BLK, 128), lambda g: (g, 0)),
        out_shape=jax.ShapeDtypeStruct((grouped.shape[0] // NS, 128),
                                       jnp.float32),
    )(grouped, nxf, w_xyz, b0, w1, b1, w2, b2)


def kernel(xyz, points, W0, b0, W1, b1, W2, b2):
    xt = jnp.transpose(xyz, (2, 0, 1))
    nx = _fps_call(xt)
    new_xyz = jnp.transpose(nx, (1, 2, 0))

    table = _t0_call(points, xyz, W0[3:], W0[:3])
    nxf = new_xyz.reshape(B * S, 3)
    b0r = b0.reshape(1, -1)
    b1r = b1.reshape(1, -1)
    b2r = b2.reshape(1, -1)

    halves = []
    hb = B // 4
    hrows = hb * S * NS
    for h in range(4):
        gidx = _bq_call(xt, new_xyz, h * hb, hb)
        idx3 = gidx.reshape(SC_WORKERS, hrows // (SC_WORKERS * CHUNK), CHUNK)
        grouped = _sc_gather_call(idx3, table)
        nslice = nxf[h * hb * S:(h + 1) * hb * S]
        halves.append(_mlp_call(grouped, nslice, W0[:3], b0r, W1, b1r, W2,
                                b2r))
    pooled = jnp.concatenate(halves, axis=0)
    return new_xyz, pooled.reshape(B, S, 128)


# speedup vs baseline: 1.0767x; 1.0767x over previous
import functools

import jax
import jax.numpy as jnp
from jax import lax
from jax.experimental import pallas as pl
from jax.experimental.pallas import tpu as pltpu
from jax.experimental.pallas import tpu_sc as plsc

B = 8
N = 2048
C = 64
S = 512
NS = 32
RADIUS2 = 0.2 ** 2
H0 = 64

SC_CORES = 2
SC_SUBCORES = 16
SC_WORKERS = SC_CORES * SC_SUBCORES
ROWS_TOTAL = B * S * NS
ROWS_PER_W = ROWS_TOTAL // SC_WORKERS
CHUNK = 128
CHUNKS_PER_W = ROWS_PER_W // CHUNK


def _row_sum(a):
    w = a.shape[1]
    while w > 128:
        h = w // 2
        a = a[:, :h] + a[:, h:]
        w = h
    return jnp.sum(a, axis=1, keepdims=True)


def _row_max(a):
    w = a.shape[1]
    while w > 128:
        h = w // 2
        a = jnp.maximum(a[:, :h], a[:, h:])
        w = h
    return jnp.max(a, axis=1, keepdims=True)


def _row_min(a):
    w = a.shape[1]
    while w > 128:
        h = w // 2
        a = jnp.minimum(a[:, :h], a[:, h:])
        w = h
    return jnp.min(a, axis=1, keepdims=True)


def _fps_body(xt_ref, nx_ref):
    x = xt_ref[0]
    y = xt_ref[1]
    z = xt_ref[2]
    col = lax.broadcasted_iota(jnp.int32, (B, N), 1)
    col_s = lax.broadcasted_iota(jnp.int32, (B, S), 1)

    def step(i, carry):
        distance, far, ax, ay, az = carry
        onehot = col == far
        cx = _row_sum(jnp.where(onehot, x, 0.0))
        cy = _row_sum(jnp.where(onehot, y, 0.0))
        cz = _row_sum(jnp.where(onehot, z, 0.0))
        sel = col_s == i
        ax = jnp.where(sel, cx, ax)
        ay = jnp.where(sel, cy, ay)
        az = jnp.where(sel, cz, az)
        dx = x - cx
        dy = y - cy
        dz = z - cz
        dist = dx * dx + dy * dy + dz * dz
        distance = jnp.minimum(distance, dist)
        rowmax = _row_max(distance)
        far = _row_min(jnp.where(distance == rowmax, col, N)).astype(jnp.int32)
        return distance, far, ax, ay, az

    init = (jnp.full((B, N), 1e10, dtype=jnp.float32),
            jnp.zeros((B, 1), dtype=jnp.int32),
            jnp.zeros((B, S), dtype=jnp.float32),
            jnp.zeros((B, S), dtype=jnp.float32),
            jnp.zeros((B, S), dtype=jnp.float32))
    _, _, ax, ay, az = lax.fori_loop(0, S, step, init)
    nx_ref[0] = ax
    nx_ref[1] = ay
    nx_ref[2] = az


def _fps_call(xt):
    return pl.pallas_call(
        _fps_body,
        grid=(1,),
        in_specs=[pl.BlockSpec((3, B, N), lambda i: (0, 0, 0))],
        out_specs=pl.BlockSpec((3, B, S), lambda i: (0, 0, 0)),
        out_shape=jax.ShapeDtypeStruct((3, B, S), jnp.float32),
    )(xt)


S_BLK = 512


def _bq_body(b0, xt_ref, nxz_ref, out_ref):
    b = pl.program_id(0) + b0
    px = xt_ref[0, pl.dslice(b, 1), :]
    py = xt_ref[1, pl.dslice(b, 1), :]
    pz = xt_ref[2, pl.dslice(b, 1), :]
    nblk = nxz_ref[0]
    cx = nblk[:, 0:1]
    cy = nblk[:, 1:2]
    cz = nblk[:, 2:3]
    dx = px - cx
    dy = py - cy
    dz = pz - cz
    d2 = dx * dx + dy * dy + dz * dz
    col = lax.broadcasted_iota(jnp.int32, (S_BLK, N), 1)
    cand_base = jnp.where(d2 <= RADIUS2, col, N)
    base = b * N

    def row_min_native(a):
        w = a.shape[1]
        while w > 128:
            h = w // 2
            a = jnp.minimum(a[:, :h], a[:, h:])
            w = h
        return jnp.min(a, axis=1, keepdims=True)

    first = row_min_native(cand_base)
    out_ref[0, :, 0:1] = first + base
    prev = first
    for k in range(1, NS):
        cand = jnp.where(col > prev, cand_base, N)
        m = row_min_native(cand)
        out_ref[0, :, k:k + 1] = jnp.where(m == N, first, m) + base
        prev = m


def _bq_call(xt, new_xyz, b0, nb):
    return pl.pallas_call(
        functools.partial(_bq_body, b0),
        grid=(nb, S // S_BLK),
        in_specs=[
            pl.BlockSpec((3, B, N), lambda b, sb: (0, 0, 0)),
            pl.BlockSpec((1, S_BLK, 3), lambda b, sb: (b + b0, sb, 0)),
        ],
        out_specs=pl.BlockSpec((1, S_BLK, NS), lambda b, sb: (b, sb, 0)),
        out_shape=jax.ShapeDtypeStruct((nb, S, NS), jnp.int32),
    )(xt, new_xyz)


def _t0_body(p_ref, x_ref, wp_ref, wx_ref, out_ref):
    out_ref[...] = (
        jnp.dot(p_ref[0], wp_ref[...], preferred_element_type=jnp.float32)
        + jnp.dot(x_ref[0], wx_ref[...], preferred_element_type=jnp.float32))


def _t0_call(points, xyz, w_pts, w_xyz):
    return pl.pallas_call(
        _t0_body,
        grid=(B,),
        in_specs=[
            pl.BlockSpec((1, N, C), lambda b: (b, 0, 0)),
            pl.BlockSpec((1, N, 3), lambda b: (b, 0, 0)),
            pl.BlockSpec((C, H0), lambda b: (0, 0)),
            pl.BlockSpec((3, H0), lambda b: (0, 0)),
        ],
        out_specs=pl.BlockSpec((N, H0), lambda b: (b, 0)),
        out_shape=jax.ShapeDtypeStruct((B * N, H0), jnp.float32),
    )(points, xyz, w_pts, w_xyz)


def _sc_gather_call(idx3, table):
    nchunks = idx3.shape[1]
    nrows = SC_WORKERS * nchunks * CHUNK

    @functools.partial(
        pl.kernel,
        out_type=jax.ShapeDtypeStruct((nrows, H0), jnp.float32),
        mesh=plsc.VectorSubcoreMesh(core_axis_name="c", subcore_axis_name="s"),
        compiler_params=pltpu.CompilerParams(use_tc_tiling_on_sc=False),
        scratch_types=[
            pltpu.VMEM((nchunks, CHUNK), jnp.int32),
            pltpu.VMEM((CHUNK, H0), jnp.float32),
            pltpu.VMEM((CHUNK, H0), jnp.float32),
            pltpu.SemaphoreType.DMA,
            pltpu.SemaphoreType.DMA,
        ],
    )
    def k(idx_hbm, table_hbm, out_hbm, idx_v, buf0, buf1, sem0, sem1):
        wid = lax.axis_index("s") * SC_CORES + lax.axis_index("c")
        pltpu.sync_copy(idx_hbm.at[wid], idx_v)
        row0 = wid * nchunks * CHUNK

        pltpu.async_copy(table_hbm.at[idx_v.at[0]], buf0, sem0)

        def pair_body(jh, _):
            j0 = 2 * jh
            pltpu.async_copy(table_hbm.at[idx_v.at[j0 + 1]], buf1, sem1)
            pltpu.make_async_copy(table_hbm.at[pl.dslice(0, CHUNK)], buf0,
                                  sem0).wait()
            pltpu.sync_copy(buf0, out_hbm.at[pl.dslice(row0 + j0 * CHUNK,
                                                       CHUNK)])

            @pl.when(jh < nchunks // 2 - 1)
            def _():
                pltpu.async_copy(table_hbm.at[idx_v.at[j0 + 2]], buf0, sem0)

            pltpu.make_async_copy(table_hbm.at[pl.dslice(0, CHUNK)], buf1,
                                  sem1).wait()
            pltpu.sync_copy(buf1, out_hbm.at[pl.dslice(row0 + (j0 + 1) * CHUNK,
                                                       CHUNK)])
            return 0

        lax.fori_loop(0, nchunks // 2, pair_body, 0)

    return k(idx3, table)


M_BLK = 128
MR = M_BLK * NS


def _mlp_body(g_ref, nxz_ref, wx_ref, b0_ref, w1_ref, b1_ref, w2_ref, b2_ref,
              out_ref):
    xr = g_ref[...]
    nxz = nxz_ref[...]
    corr = jnp.dot(nxz, wx_ref[...],
                   preferred_element_type=jnp.float32) - b0_ref[...]
    h = (xr.reshape(M_BLK, NS, H0) - corr[:, None, :]).reshape(MR, H0)
    h = jnp.maximum(h, 0.0)
    h = jnp.dot(h, w1_ref[...], preferred_element_type=jnp.float32)
    h = jnp.maximum(h + b1_ref[...], 0.0)
    h = jnp.dot(h, w2_ref[...], preferred_element_type=jnp.float32)
    h = jnp.maximum(h + b2_ref[...], 0.0)
    out_ref[...] = jnp.max(h.reshape(M_BLK, NS, 128), axis=1)


def _mlp_call(grouped, nxf, w_xyz, b0, w1, b1, w2, b2):
    nblk = grouped.shape[0] // MR
    return pl.pallas_call(
        _mlp_body,
        grid=(nblk,),
        in_specs=[
            pl.BlockSpec((MR, H0), lambda g: (g, 0)),
            pl.BlockSpec((M_BLK, 3), lambda g: (g, 0)),
            pl.BlockSpec((3, H0), lambda g: (0, 0)),
            pl.BlockSpec((1, 64), lambda g: (0, 0)),
            pl.BlockSpec((64, 64), lambda g: (0, 0)),
            pl.BlockSpec((1, 64), lambda g: (0, 0)),
            pl.BlockSpec((64, 128), lambda g: (0, 0)),
            pl.BlockSpec((1, 128), lambda g: (0, 0)),
        ],
        out_specs=pl.BlockSpec((M_BLK, 128), lambda g: (g, 0)),
        out_shape=jax.ShapeDtypeStruct((grouped.shape[0] // NS, 128),
                                       jnp.float32),
    )(grouped, nxf, w_xyz, b0, w1, b1, w2, b2)


def kernel(xyz, points, W0, b0, W1, b1, W2, b2):
    xt = jnp.transpose(xyz, (2, 0, 1))
    nx = _fps_call(xt)
    new_xyz = jnp.transpose(nx, (1, 2, 0))

    table = _t0_call(points, xyz, W0[3:], W0[:3])
    nxf = new_xyz.reshape(B * S, 3)
    b0r = b0.reshape(1, -1)
    b1r = b1.reshape(1, -1)
    b2r = b2.reshape(1, -1)

    halves = []
    hb = B // 2
    hrows = hb * S * NS
    for h in range(2):
        gidx = _bq_call(xt, new_xyz, h * hb, hb)
        idx3 = gidx.reshape(SC_WORKERS, hrows // (SC_WORKERS * CHUNK), CHUNK)
        grouped = _sc_gather_call(idx3, table)
        nslice = nxf[h * hb * S:(h + 1) * hb * S]
        halves.append(_mlp_call(grouped, nslice, W0[:3], b0r, W1, b1r, W2,
                                b2r))
    pooled = jnp.concatenate(halves, axis=0)
    return new_xyz, pooled.reshape(B, S, 128)
